# baseline (speedup 1.0000x reference)
import numpy as np

import jax
import jax.numpy as jnp
from jax.experimental import pallas as pl

B = 2048
D = 768
M = 100
K = 5
H1 = 512
H2 = 256
OUT = 4
P = 64 * 64

RA = 768
BB = 256


def _mut_expr():
    k1, k2 = jax.random.split(jax.random.key(42))
    mask = (jax.random.uniform(k1, (B, K, D)) < 0.1).astype(jnp.float32)
    noise = jax.random.normal(k2, (B, K, D), dtype=jnp.float32) * 0.05
    return (mask * noise).reshape(B, K * D).astype(jnp.bfloat16)


_MUT_CACHE = {}


def _mut_flat():
    if "v" not in _MUT_CACHE:
        try:
            with jax.ensure_compile_time_eval():
                _MUT_CACHE["v"] = np.asarray(_mut_expr())
        except Exception:
            try:
                with jax.default_device(jax.devices("cpu")[0]):
                    _MUT_CACHE["v"] = np.asarray(_mut_expr())
            except Exception:
                return None
    return _MUT_CACHE["v"]


def _phase_a(img_ref, means_ref):
    x = img_ref[...]
    means_ref[...] = jnp.sum(x, axis=-1, keepdims=True) * (1.0 / P)


def _phase_b(mblk_ref, mfull_ref, wpt_ref, bp_ref, memory_ref, mut_ref,
             w1_ref, b1_ref, w2_ref, b2_ref, w3_ref, b3_ref, out_ref):
    wpt_bf = wpt_ref[...]
    enc = jax.lax.dot_general(mblk_ref[...].astype(jnp.bfloat16), wpt_bf,
                              (((1,), (0,)), ((), ())),
                              preferred_element_type=jnp.float32)
    enc = enc + bp_ref[...]

    msum = jnp.sum(mfull_ref[...].astype(jnp.bfloat16).astype(jnp.float32),
                   axis=0, keepdims=True)
    enc_mean = jax.lax.dot_general(msum, wpt_bf.astype(jnp.float32),
                                   (((1,), (0,)), ((), ())),
                                   precision=jax.lax.Precision.HIGHEST,
                                   preferred_element_type=jnp.float32)
    enc_mean = enc_mean * (1.0 / B) + bp_ref[...]
    row = jax.lax.broadcasted_iota(jnp.int32, (M, D), 0)
    mem = jnp.where(row == 0, enc_mean, memory_ref[...])

    num = jax.lax.dot_general(enc, mem, (((1,), (1,)), ((), ())),
                              precision=jax.lax.Precision.HIGHEST,
                              preferred_element_type=jnp.float32)
    enc_n = jnp.sqrt(jnp.sum(enc * enc, axis=1, keepdims=True))
    ones = jnp.ones((1, D), dtype=jnp.float32)
    mem_n2 = jax.lax.dot_general(ones, mem * mem, (((1,), (1,)), ((), ())),
                                 precision=jax.lax.Precision.HIGHEST,
                                 preferred_element_type=jnp.float32)
    mem_n = jnp.sqrt(mem_n2)
    sim = num / jnp.maximum(enc_n * mem_n, 1e-8)

    col = jax.lax.broadcasted_iota(jnp.int32, (BB, M), 1)
    w1 = w1_ref[...]
    acc = jax.lax.dot_general(enc.astype(jnp.bfloat16), w1[:, :D],
                              (((1,), (1,)), ((), ())),
                              preferred_element_type=jnp.float32)
    work = sim
    for k in range(K):
        mx = jnp.max(work, axis=1, keepdims=True)
        idxk = jnp.min(jnp.where(work == mx, col, jnp.int32(1 << 30)),
                       axis=1, keepdims=True)
        onehot = (col == idxk).astype(jnp.float32)
        selk = jax.lax.dot_general(onehot, mem, (((1,), (0,)), ((), ())),
                                   precision=jax.lax.Precision.HIGHEST,
                                   preferred_element_type=jnp.float32)
        chunk = (selk + mut_ref[:, k * D:(k + 1) * D].astype(jnp.float32)
                 ).astype(jnp.bfloat16)
        acc = acc + jax.lax.dot_general(
            chunk, w1[:, (k + 1) * D:(k + 2) * D], (((1,), (1,)), ((), ())),
            preferred_element_type=jnp.float32)
        work = jnp.where(col == idxk, jnp.float32(-jnp.inf), work)

    h = jax.nn.relu(acc + b1_ref[...])
    h = jax.lax.dot_general(h.astype(jnp.bfloat16), w2_ref[...],
                            (((1,), (1,)), ((), ())),
                            preferred_element_type=jnp.float32)
    h = jax.nn.relu(h + b2_ref[...])
    o = jax.lax.dot_general(h.astype(jnp.bfloat16), w3_ref[...],
                            (((1,), (1,)), ((), ())),
                            preferred_element_type=jnp.float32)
    out_ref[...] = o + b3_ref[...]


def kernel(image_stream, W_proj, b_proj, memory, W1, b1, W2, b2, W3, b3):
    wpt = W_proj.T.astype(jnp.bfloat16)
    bp = b_proj.reshape(1, D)
    w1b = W1.astype(jnp.bfloat16)
    w2b = W2.astype(jnp.bfloat16)
    w3b = W3.astype(jnp.bfloat16)
    mut_np = _mut_flat()
    mut = jnp.asarray(mut_np) if mut_np is not None else _mut_expr()

    latest = image_stream[:, 1].reshape(B * 3, P)

    sums = pl.pallas_call(
        _phase_a,
        grid=(B * 3 // RA,),
        in_specs=[pl.BlockSpec((RA, P), lambda i: (i, 0))],
        out_specs=pl.BlockSpec((RA, 1), lambda i: (i, 0)),
        out_shape=jax.ShapeDtypeStruct((B * 3, 1), jnp.float32),
    )(latest)
    means = sums.reshape(B, 3)

    out = pl.pallas_call(
        _phase_b,
        grid=(B // BB,),
        in_specs=[
            pl.BlockSpec((BB, 3), lambda i: (i, 0)),
            pl.BlockSpec((B, 3), lambda i: (0, 0)),
            pl.BlockSpec((3, D), lambda i: (0, 0)),
            pl.BlockSpec((1, D), lambda i: (0, 0)),
            pl.BlockSpec((M, D), lambda i: (0, 0)),
            pl.BlockSpec((BB, K * D), lambda i: (i, 0)),
            pl.BlockSpec((H1, 6 * D), lambda i: (0, 0)),
            pl.BlockSpec((1, H1), lambda i: (0, 0)),
            pl.BlockSpec((H2, H1), lambda i: (0, 0)),
            pl.BlockSpec((1, H2), lambda i: (0, 0)),
            pl.BlockSpec((OUT, H2), lambda i: (0, 0)),
            pl.BlockSpec((1, OUT), lambda i: (0, 0)),
        ],
        out_specs=pl.BlockSpec((BB, OUT), lambda i: (i, 0)),
        out_shape=jax.ShapeDtypeStruct((B, OUT), jnp.float32),
    )(means, means, wpt, bp, memory, mut, w1b, b1.reshape(1, H1), w2b,
      b2.reshape(1, H2), w3b, b3.reshape(1, OUT))
    return out


# speedup vs baseline: 1.6181x; 1.6181x over previous
import numpy as np

import jax
import jax.numpy as jnp
from jax.experimental import pallas as pl

B = 2048
D = 768
M = 100
K = 5
H1 = 512
H2 = 256
OUT = 4
P = 64 * 64

BA = 128
BB = 256


def _mut_expr():
    k1, k2 = jax.random.split(jax.random.key(42))
    mask = (jax.random.uniform(k1, (B, K, D)) < 0.1).astype(jnp.float32)
    noise = jax.random.normal(k2, (B, K, D), dtype=jnp.float32) * 0.05
    return (mask * noise).reshape(B, K * D).astype(jnp.bfloat16)


_MUT_CACHE = {}


def _mut_flat():
    if "v" not in _MUT_CACHE:
        try:
            with jax.ensure_compile_time_eval():
                _MUT_CACHE["v"] = np.asarray(_mut_expr())
        except Exception:
            try:
                with jax.default_device(jax.devices("cpu")[0]):
                    _MUT_CACHE["v"] = np.asarray(_mut_expr())
            except Exception:
                return None
    return _MUT_CACHE["v"]


def _phase_a(img_ref, means_ref):
    x = img_ref[...]
    for c in range(3):
        means_ref[:, c:c + 1] = jnp.sum(
            x[:, c * P:(c + 1) * P], axis=-1, keepdims=True) * (1.0 / P)


def _phase_b(mblk_ref, mfull_ref, wpt_ref, bp_ref, memory_ref, mut_ref,
             w1_ref, b1_ref, w2_ref, b2_ref, w3_ref, b3_ref, out_ref):
    wpt_bf = wpt_ref[...]
    enc = jax.lax.dot_general(mblk_ref[...].astype(jnp.bfloat16), wpt_bf,
                              (((1,), (0,)), ((), ())),
                              preferred_element_type=jnp.float32)
    enc = enc + bp_ref[...]

    msum = jnp.sum(mfull_ref[...].astype(jnp.bfloat16).astype(jnp.float32),
                   axis=0, keepdims=True)
    enc_mean = jax.lax.dot_general(msum, wpt_bf.astype(jnp.float32),
                                   (((1,), (0,)), ((), ())),
                                   precision=jax.lax.Precision.HIGHEST,
                                   preferred_element_type=jnp.float32)
    enc_mean = enc_mean * (1.0 / B) + bp_ref[...]
    row = jax.lax.broadcasted_iota(jnp.int32, (M, D), 0)
    mem = jnp.where(row == 0, enc_mean, memory_ref[...])

    num = jax.lax.dot_general(enc, mem, (((1,), (1,)), ((), ())),
                              precision=jax.lax.Precision.HIGHEST,
                              preferred_element_type=jnp.float32)
    enc_n = jnp.sqrt(jnp.sum(enc * enc, axis=1, keepdims=True))
    ones = jnp.ones((1, D), dtype=jnp.float32)
    mem_n2 = jax.lax.dot_general(ones, mem * mem, (((1,), (1,)), ((), ())),
                                 precision=jax.lax.Precision.HIGHEST,
                                 preferred_element_type=jnp.float32)
    mem_n = jnp.sqrt(mem_n2)
    sim = num / jnp.maximum(enc_n * mem_n, 1e-8)

    col = jax.lax.broadcasted_iota(jnp.int32, (BB, M), 1)
    w1 = w1_ref[...]
    acc = jax.lax.dot_general(enc.astype(jnp.bfloat16), w1[:, :D],
                              (((1,), (1,)), ((), ())),
                              preferred_element_type=jnp.float32)
    work = sim
    for k in range(K):
        mx = jnp.max(work, axis=1, keepdims=True)
        idxk = jnp.min(jnp.where(work == mx, col, jnp.int32(1 << 30)),
                       axis=1, keepdims=True)
        onehot = (col == idxk).astype(jnp.float32)
        selk = jax.lax.dot_general(onehot, mem, (((1,), (0,)), ((), ())),
                                   precision=jax.lax.Precision.HIGHEST,
                                   preferred_element_type=jnp.float32)
        chunk = (selk + mut_ref[:, k * D:(k + 1) * D].astype(jnp.float32)
                 ).astype(jnp.bfloat16)
        acc = acc + jax.lax.dot_general(
            chunk, w1[:, (k + 1) * D:(k + 2) * D], (((1,), (1,)), ((), ())),
            preferred_element_type=jnp.float32)
        work = jnp.where(col == idxk, jnp.float32(-jnp.inf), work)

    h = jax.nn.relu(acc + b1_ref[...])
    h = jax.lax.dot_general(h.astype(jnp.bfloat16), w2_ref[...],
                            (((1,), (1,)), ((), ())),
                            preferred_element_type=jnp.float32)
    h = jax.nn.relu(h + b2_ref[...])
    o = jax.lax.dot_general(h.astype(jnp.bfloat16), w3_ref[...],
                            (((1,), (1,)), ((), ())),
                            preferred_element_type=jnp.float32)
    out_ref[...] = o + b3_ref[...]


def kernel(image_stream, W_proj, b_proj, memory, W1, b1, W2, b2, W3, b3):
    wpt = W_proj.T.astype(jnp.bfloat16)
    bp = b_proj.reshape(1, D)
    w1b = W1.astype(jnp.bfloat16)
    w2b = W2.astype(jnp.bfloat16)
    w3b = W3.astype(jnp.bfloat16)
    mut_np = _mut_flat()
    mut = jnp.asarray(mut_np) if mut_np is not None else _mut_expr()

    latest = image_stream[:, 1].reshape(B, 3 * P)

    means = pl.pallas_call(
        _phase_a,
        grid=(B // BA,),
        in_specs=[pl.BlockSpec((BA, 3 * P), lambda i: (i, 0))],
        out_specs=pl.BlockSpec((BA, 3), lambda i: (i, 0)),
        out_shape=jax.ShapeDtypeStruct((B, 3), jnp.float32),
    )(latest)

    out = pl.pallas_call(
        _phase_b,
        grid=(B // BB,),
        in_specs=[
            pl.BlockSpec((BB, 3), lambda i: (i, 0)),
            pl.BlockSpec((B, 3), lambda i: (0, 0)),
            pl.BlockSpec((3, D), lambda i: (0, 0)),
            pl.BlockSpec((1, D), lambda i: (0, 0)),
            pl.BlockSpec((M, D), lambda i: (0, 0)),
            pl.BlockSpec((BB, K * D), lambda i: (i, 0)),
            pl.BlockSpec((H1, 6 * D), lambda i: (0, 0)),
            pl.BlockSpec((1, H1), lambda i: (0, 0)),
            pl.BlockSpec((H2, H1), lambda i: (0, 0)),
            pl.BlockSpec((1, H2), lambda i: (0, 0)),
            pl.BlockSpec((OUT, H2), lambda i: (0, 0)),
            pl.BlockSpec((1, OUT), lambda i: (0, 0)),
        ],
        out_specs=pl.BlockSpec((BB, OUT), lambda i: (i, 0)),
        out_shape=jax.ShapeDtypeStruct((B, OUT), jnp.float32),
    )(means, means, wpt, bp, memory, mut, w1b, b1.reshape(1, H1), w2b,
      b2.reshape(1, H2), w3b, b3.reshape(1, OUT))
    return out


# speedup vs baseline: 1.7464x; 1.0793x over previous
import numpy as np

import jax
import jax.numpy as jnp
from jax.experimental import pallas as pl

B = 2048
D = 768
M = 100
K = 5
H1 = 512
H2 = 256
OUT = 4
P = 64 * 64

BA = 128
BB = 256


def _mut_expr():
    k1, k2 = jax.random.split(jax.random.key(42))
    mask = (jax.random.uniform(k1, (B, K, D)) < 0.1).astype(jnp.float32)
    noise = jax.random.normal(k2, (B, K, D), dtype=jnp.float32) * 0.05
    return (mask * noise).reshape(B, K * D).astype(jnp.bfloat16)


_MUT_CACHE = {}


def _mut_flat():
    if "v" not in _MUT_CACHE:
        try:
            with jax.ensure_compile_time_eval():
                _MUT_CACHE["v"] = np.asarray(_mut_expr())
        except Exception:
            try:
                with jax.default_device(jax.devices("cpu")[0]):
                    _MUT_CACHE["v"] = np.asarray(_mut_expr())
            except Exception:
                return None
    return _MUT_CACHE["v"]


def _phase_a(img_ref, means_ref):
    x = img_ref[...]
    for c in range(3):
        means_ref[:, c:c + 1] = jnp.sum(
            x[:, c * P:(c + 1) * P], axis=-1, keepdims=True) * (1.0 / P)


def _phase_b(mblk_ref, mfull_ref, wpt_ref, bp_ref, memory_ref, mut_ref,
             w1_ref, b1_ref, w2_ref, b2_ref, w3_ref, b3_ref, out_ref):
    wpt_bf = wpt_ref[...]
    enc = jax.lax.dot_general(mblk_ref[...].astype(jnp.bfloat16), wpt_bf,
                              (((1,), (0,)), ((), ())),
                              preferred_element_type=jnp.float32)
    enc = enc + bp_ref[...]

    msum = jnp.sum(mfull_ref[...].astype(jnp.bfloat16).astype(jnp.float32),
                   axis=0, keepdims=True)
    enc_mean = jax.lax.dot_general(msum, wpt_bf.astype(jnp.float32),
                                   (((1,), (0,)), ((), ())),
                                   precision=jax.lax.Precision.HIGHEST,
                                   preferred_element_type=jnp.float32)
    enc_mean = enc_mean * (1.0 / B) + bp_ref[...]
    row = jax.lax.broadcasted_iota(jnp.int32, (M, D), 0)
    mem = jnp.where(row == 0, enc_mean, memory_ref[...])

    num = jax.lax.dot_general(enc, mem, (((1,), (1,)), ((), ())),
                              precision=jax.lax.Precision.HIGHEST,
                              preferred_element_type=jnp.float32)
    enc_n = jnp.sqrt(jnp.sum(enc * enc, axis=1, keepdims=True))
    ones = jnp.ones((1, D), dtype=jnp.float32)
    mem_n2 = jax.lax.dot_general(ones, mem * mem, (((1,), (1,)), ((), ())),
                                 precision=jax.lax.Precision.HIGHEST,
                                 preferred_element_type=jnp.float32)
    mem_n = jnp.sqrt(mem_n2)
    sim = num / jnp.maximum(enc_n * mem_n, 1e-8)

    col = jax.lax.broadcasted_iota(jnp.int32, (BB, M), 1)
    w1 = w1_ref[...]
    mem_bf = mem.astype(jnp.bfloat16)
    acc = jax.lax.dot_general(enc.astype(jnp.bfloat16), w1[:, :D],
                              (((1,), (1,)), ((), ())),
                              preferred_element_type=jnp.float32)
    work = sim
    for k in range(K):
        mx = jnp.max(work, axis=1, keepdims=True)
        idxk = jnp.min(jnp.where(work == mx, col, jnp.int32(1 << 30)),
                       axis=1, keepdims=True)
        onehot = (col == idxk).astype(jnp.bfloat16)
        selk = jax.lax.dot_general(onehot, mem_bf, (((1,), (0,)), ((), ())),
                                   preferred_element_type=jnp.float32)
        chunk = (selk + mut_ref[:, k * D:(k + 1) * D].astype(jnp.float32)
                 ).astype(jnp.bfloat16)
        acc = acc + jax.lax.dot_general(
            chunk, w1[:, (k + 1) * D:(k + 2) * D], (((1,), (1,)), ((), ())),
            preferred_element_type=jnp.float32)
        work = jnp.where(col == idxk, jnp.float32(-jnp.inf), work)

    h = jax.nn.relu(acc + b1_ref[...])
    h = jax.lax.dot_general(h.astype(jnp.bfloat16), w2_ref[...],
                            (((1,), (1,)), ((), ())),
                            preferred_element_type=jnp.float32)
    h = jax.nn.relu(h + b2_ref[...])
    o = jax.lax.dot_general(h.astype(jnp.bfloat16), w3_ref[...],
                            (((1,), (1,)), ((), ())),
                            preferred_element_type=jnp.float32)
    out_ref[...] = o + b3_ref[...]


def kernel(image_stream, W_proj, b_proj, memory, W1, b1, W2, b2, W3, b3):
    wpt = W_proj.T.astype(jnp.bfloat16)
    bp = b_proj.reshape(1, D)
    w1b = W1.astype(jnp.bfloat16)
    w2b = W2.astype(jnp.bfloat16)
    w3b = W3.astype(jnp.bfloat16)
    mut_np = _mut_flat()
    mut = jnp.asarray(mut_np) if mut_np is not None else _mut_expr()

    latest = image_stream[:, 1].reshape(B, 3 * P)

    means = pl.pallas_call(
        _phase_a,
        grid=(B // BA,),
        in_specs=[pl.BlockSpec((BA, 3 * P), lambda i: (i, 0))],
        out_specs=pl.BlockSpec((BA, 3), lambda i: (i, 0)),
        out_shape=jax.ShapeDtypeStruct((B, 3), jnp.float32),
    )(latest)

    out = pl.pallas_call(
        _phase_b,
        grid=(B // BB,),
        in_specs=[
            pl.BlockSpec((BB, 3), lambda i: (i, 0)),
            pl.BlockSpec((B, 3), lambda i: (0, 0)),
            pl.BlockSpec((3, D), lambda i: (0, 0)),
            pl.BlockSpec((1, D), lambda i: (0, 0)),
            pl.BlockSpec((M, D), lambda i: (0, 0)),
            pl.BlockSpec((BB, K * D), lambda i: (i, 0)),
            pl.BlockSpec((H1, 6 * D), lambda i: (0, 0)),
            pl.BlockSpec((1, H1), lambda i: (0, 0)),
            pl.BlockSpec((H2, H1), lambda i: (0, 0)),
            pl.BlockSpec((1, H2), lambda i: (0, 0)),
            pl.BlockSpec((OUT, H2), lambda i: (0, 0)),
            pl.BlockSpec((1, OUT), lambda i: (0, 0)),
        ],
        out_specs=pl.BlockSpec((BB, OUT), lambda i: (i, 0)),
        out_shape=jax.ShapeDtypeStruct((B, OUT), jnp.float32),
    )(means, means, wpt, bp, memory, mut, w1b, b1.reshape(1, H1), w2b,
      b2.reshape(1, H2), w3b, b3.reshape(1, OUT))
    return out
